# baseline (speedup 1.0000x reference)
import functools

import jax
import jax.numpy as jnp
from jax import lax
from jax.experimental import pallas as pl
from jax.experimental.pallas import tpu as pltpu
from jax.experimental.pallas import tpu_sc as plsc

N_POINTS = 320000
D = 128
BLK = 128
NBLKS = N_POINTS // BLK
NW = 32
BPW = -(-NBLKS // NW)
BPW_PAD = 80
NBUF = 4
ROWS_PER_TILE = 72
ACC_ROWS = 16 * ROWS_PER_TILE
NSEG = 1024
CW = 128


def _sc_body(data_hbm, ids_hbm, zeros_hbm, ones_hbm, sums_hbm, cnts_hbm,
             gidx_v, rows_v, idxrow_v, dbuf_v, ones_v, zbuf_v, czbuf_v,
             acc_sh, cacc_sh, gsem, ssem, osem, isem):
    c = lax.axis_index("c")
    s = lax.axis_index("s")
    w = s * 2 + c

    for k in range(BPW_PAD // 16):
        jv = jnp.arange(16, dtype=jnp.int32) + 16 * k
        gidx_v[pl.ds(16 * k, 16)] = jnp.minimum(w + NW * jv, NBLKS - 1)

    base = ROWS_PER_TILE * s
    pltpu.async_copy(ids_hbm.at[gidx_v], rows_v, isem)
    pltpu.async_copy(ones_hbm, ones_v, gsem)
    pltpu.async_copy(zeros_hbm.at[pl.ds(0, ROWS_PER_TILE)],
                     acc_sh.at[pl.ds(base, ROWS_PER_TILE)], ssem)
    pltpu.async_copy(zeros_hbm.at[pl.ds(0, ROWS_PER_TILE)],
                     cacc_sh.at[pl.ds(base, ROWS_PER_TILE)], osem)
    pltpu.make_async_copy(ones_hbm, ones_v, gsem).wait()
    pltpu.make_async_copy(zeros_hbm.at[pl.ds(0, ROWS_PER_TILE)],
                          acc_sh.at[pl.ds(base, ROWS_PER_TILE)], ssem).wait()
    pltpu.make_async_copy(zeros_hbm.at[pl.ds(0, ROWS_PER_TILE)],
                          cacc_sh.at[pl.ds(base, ROWS_PER_TILE)], osem).wait()
    pltpu.make_async_copy(ids_hbm.at[gidx_v], rows_v, isem).wait()

    def _count(j, r):
        b = w + NW * j
        first = rows_v[j, pl.ds(0, 16)][0]
        ok = jnp.logical_and(b < NBLKS, first < NSEG)
        return r + jnp.where(ok, 1, 0).astype(jnp.int32)

    nrel = lax.fori_loop(0, BPW, _count, jnp.int32(0))

    plsc.subcore_barrier()

    def _outer(jo, carry):
        j0 = jo * NBUF
        for t in range(NBUF):
            @pl.when(j0 + t < nrel)
            def _():
                b = w + NW * (j0 + t)
                pltpu.async_copy(
                    data_hbm.at[pl.ds(b * BLK, BLK)], dbuf_v.at[t], gsem)
        for t in range(NBUF):
            @pl.when(j0 + t < nrel)
            def _():
                j = j0 + t
                b = w + NW * j
                pltpu.make_async_copy(
                    data_hbm.at[pl.ds(b * BLK, BLK)], dbuf_v.at[t], gsem
                ).wait()
                for cc in range(BLK // 16):
                    v = rows_v[j, pl.ds(16 * cc, 16)]
                    idxrow_v[t, pl.ds(16 * cc, 16)] = jnp.minimum(v, NSEG)
                pltpu.async_copy(
                    dbuf_v.at[t], acc_sh.at[idxrow_v.at[t]], ssem, add=True)
                pltpu.async_copy(
                    ones_v, cacc_sh.at[idxrow_v.at[t]], osem, add=True)
        for t in range(NBUF):
            @pl.when(j0 + t < nrel)
            def _():
                pltpu.make_async_copy(
                    dbuf_v.at[t], acc_sh.at[idxrow_v.at[t]], ssem).wait()
                pltpu.make_async_copy(
                    ones_v, cacc_sh.at[idxrow_v.at[t]], osem).wait()
        return carry

    lax.fori_loop(0, -(-BPW // NBUF), _outer, 0)

    plsc.subcore_barrier()

    out_base = c * ACC_ROWS + base
    pltpu.sync_copy(acc_sh.at[pl.ds(base, ROWS_PER_TILE)], zbuf_v)
    pltpu.sync_copy(zbuf_v, sums_hbm.at[pl.ds(out_base, ROWS_PER_TILE)])
    pltpu.sync_copy(cacc_sh.at[pl.ds(base, ROWS_PER_TILE)], czbuf_v)
    pltpu.sync_copy(czbuf_v, cnts_hbm.at[pl.ds(out_base, ROWS_PER_TILE)])


_sc_call = functools.partial(
    pl.kernel,
    out_type=(
        jax.ShapeDtypeStruct((2 * ACC_ROWS, D), jnp.float32),
        jax.ShapeDtypeStruct((2 * ACC_ROWS, D), jnp.float32),
    ),
    mesh=plsc.VectorSubcoreMesh(core_axis_name="c", subcore_axis_name="s"),
    scratch_types=(
        pltpu.VMEM((BPW_PAD,), jnp.int32),
        pltpu.VMEM((BPW_PAD, BLK), jnp.int32),
        pltpu.VMEM((NBUF, BLK), jnp.int32),
        pltpu.VMEM((NBUF, BLK, D), jnp.float32),
        pltpu.VMEM((BLK, CW), jnp.float32),
        pltpu.VMEM((ROWS_PER_TILE, D), jnp.float32),
        pltpu.VMEM((ROWS_PER_TILE, CW), jnp.float32),
        pltpu.VMEM_SHARED((ACC_ROWS, D), jnp.float32),
        pltpu.VMEM_SHARED((ACC_ROWS, CW), jnp.float32),
        pltpu.SemaphoreType.DMA,
        pltpu.SemaphoreType.DMA,
        pltpu.SemaphoreType.DMA,
        pltpu.SemaphoreType.DMA,
    ),
)(_sc_body)


def _finalize_body(sums_ref, cnts_ref, out_ref):
    ssum = sums_ref[0:NSEG, :] + sums_ref[ACC_ROWS:ACC_ROWS + NSEG, :]
    cnt = cnts_ref[0:NSEG, 0:1] + cnts_ref[ACC_ROWS:ACC_ROWS + NSEG, 0:1]
    out_ref[...] = ssum / jnp.maximum(cnt, 1.0)


def kernel(data, segment_ids):
    ids2d = segment_ids.astype(jnp.int32).reshape(NBLKS, BLK)
    zeros = jnp.zeros((ROWS_PER_TILE, D), jnp.float32)
    ones = jnp.ones((BLK, CW), jnp.float32)
    sums, cnts = _sc_call(data, ids2d, zeros, ones)
    return pl.pallas_call(
        _finalize_body,
        out_shape=jax.ShapeDtypeStruct((NSEG, D), jnp.float32),
    )(sums, cnts)


# speedup vs baseline: 1.0861x; 1.0861x over previous
import functools

import jax
import jax.numpy as jnp
from jax import lax
from jax.experimental import pallas as pl
from jax.experimental.pallas import tpu as pltpu
from jax.experimental.pallas import tpu_sc as plsc

N_POINTS = 320000
D = 128
BLK = 128
NBLKS = N_POINTS // BLK
NW = 32
BPW = -(-NBLKS // NW)
BPW_PAD = 80
NBUF = 5
ROWS_PER_TILE = 72
ACC_ROWS = 16 * ROWS_PER_TILE
NSEG = 1024
CW = 128


def _sc_body(data_hbm, ids_hbm, sums_hbm, cnts_hbm,
             gidx_v, rows_v, idxrow_v, dbuf_v, ones_v,
             acc_sh, cacc_sh, gsem, ssem, osem, isem):
    c = lax.axis_index("c")
    s = lax.axis_index("s")
    w = s * 2 + c

    for k in range(BPW_PAD // 16):
        jv = jnp.arange(16, dtype=jnp.int32) + 16 * k
        gidx_v[pl.ds(16 * k, 16)] = jnp.minimum(w + NW * jv, NBLKS - 1)

    pltpu.async_copy(ids_hbm.at[gidx_v], rows_v, isem)

    one16 = jnp.ones((16,), jnp.float32)
    zero16 = jnp.zeros((16,), jnp.float32)

    def _ones_row(r, carry):
        for cc in range(CW // 16):
            ones_v[r, pl.ds(16 * cc, 16)] = one16
        return carry
    lax.fori_loop(0, BLK, _ones_row, 0)

    def _zero_row(r, carry):
        for cc in range(D // 16):
            dbuf_v[0, r, pl.ds(16 * cc, 16)] = zero16
        return carry
    lax.fori_loop(0, ROWS_PER_TILE, _zero_row, 0)

    base = ROWS_PER_TILE * s
    pltpu.sync_copy(dbuf_v.at[0].at[pl.ds(0, ROWS_PER_TILE)],
                    acc_sh.at[pl.ds(base, ROWS_PER_TILE)])
    pltpu.sync_copy(dbuf_v.at[0].at[pl.ds(0, ROWS_PER_TILE)],
                    cacc_sh.at[pl.ds(base, ROWS_PER_TILE)])

    pltpu.make_async_copy(ids_hbm.at[gidx_v], rows_v, isem).wait()

    def _count(j, r):
        b = w + NW * j
        first = rows_v[j, pl.ds(0, 16)][0]
        ok = jnp.logical_and(b < NBLKS, first < NSEG)
        return r + jnp.where(ok, 1, 0).astype(jnp.int32)

    nrel = lax.fori_loop(0, BPW, _count, jnp.int32(0))

    plsc.subcore_barrier()

    def _outer(jo, carry):
        j0 = jo * NBUF
        for t in range(NBUF):
            @pl.when(j0 + t < nrel)
            def _():
                b = w + NW * (j0 + t)
                pltpu.async_copy(
                    data_hbm.at[pl.ds(b * BLK, BLK)], dbuf_v.at[t], gsem)
        for t in range(NBUF):
            @pl.when(j0 + t < nrel)
            def _():
                j = j0 + t
                b = w + NW * j
                pltpu.make_async_copy(
                    data_hbm.at[pl.ds(b * BLK, BLK)], dbuf_v.at[t], gsem
                ).wait()
                for cc in range(BLK // 16):
                    v = rows_v[j, pl.ds(16 * cc, 16)]
                    idxrow_v[t, pl.ds(16 * cc, 16)] = jnp.minimum(v, NSEG)
                pltpu.async_copy(
                    dbuf_v.at[t], acc_sh.at[idxrow_v.at[t]], ssem, add=True)
                pltpu.async_copy(
                    ones_v, cacc_sh.at[idxrow_v.at[t]], osem, add=True)
        for t in range(NBUF):
            @pl.when(j0 + t < nrel)
            def _():
                pltpu.make_async_copy(
                    dbuf_v.at[t], acc_sh.at[idxrow_v.at[t]], ssem).wait()
                pltpu.make_async_copy(
                    ones_v, cacc_sh.at[idxrow_v.at[t]], osem).wait()
        return carry

    lax.fori_loop(0, -(-BPW // NBUF), _outer, 0)

    plsc.subcore_barrier()

    out_base = c * ACC_ROWS + base
    pltpu.async_copy(acc_sh.at[pl.ds(base, ROWS_PER_TILE)],
                     sums_hbm.at[pl.ds(out_base, ROWS_PER_TILE)], gsem)
    pltpu.async_copy(cacc_sh.at[pl.ds(base, ROWS_PER_TILE)],
                     cnts_hbm.at[pl.ds(out_base, ROWS_PER_TILE)], ssem)
    pltpu.make_async_copy(acc_sh.at[pl.ds(base, ROWS_PER_TILE)],
                          sums_hbm.at[pl.ds(out_base, ROWS_PER_TILE)],
                          gsem).wait()
    pltpu.make_async_copy(cacc_sh.at[pl.ds(base, ROWS_PER_TILE)],
                          cnts_hbm.at[pl.ds(out_base, ROWS_PER_TILE)],
                          ssem).wait()


_sc_call = functools.partial(
    pl.kernel,
    out_type=(
        jax.ShapeDtypeStruct((2 * ACC_ROWS, D), jnp.float32),
        jax.ShapeDtypeStruct((2 * ACC_ROWS, CW), jnp.float32),
    ),
    mesh=plsc.VectorSubcoreMesh(core_axis_name="c", subcore_axis_name="s"),
    scratch_types=(
        pltpu.VMEM((BPW_PAD,), jnp.int32),
        pltpu.VMEM((BPW_PAD, BLK), jnp.int32),
        pltpu.VMEM((NBUF, BLK), jnp.int32),
        pltpu.VMEM((NBUF, BLK, D), jnp.float32),
        pltpu.VMEM((BLK, CW), jnp.float32),
        pltpu.VMEM_SHARED((ACC_ROWS, D), jnp.float32),
        pltpu.VMEM_SHARED((ACC_ROWS, CW), jnp.float32),
        pltpu.SemaphoreType.DMA,
        pltpu.SemaphoreType.DMA,
        pltpu.SemaphoreType.DMA,
        pltpu.SemaphoreType.DMA,
    ),
)(_sc_body)


def _finalize_body(sums_ref, cnts_ref, out_ref):
    ssum = sums_ref[0:NSEG, :] + sums_ref[ACC_ROWS:ACC_ROWS + NSEG, :]
    cnt = cnts_ref[0:NSEG, 0:1] + cnts_ref[ACC_ROWS:ACC_ROWS + NSEG, 0:1]
    out_ref[...] = ssum / jnp.maximum(cnt, 1.0)


def kernel(data, segment_ids):
    ids2d = segment_ids.astype(jnp.int32).reshape(NBLKS, BLK)
    sums, cnts = _sc_call(data, ids2d)
    return pl.pallas_call(
        _finalize_body,
        out_shape=jax.ShapeDtypeStruct((NSEG, D), jnp.float32),
    )(sums, cnts)
